# Initial kernel scaffold; baseline (speedup 1.0000x reference)
import jax
import jax.numpy as jnp
from jax.experimental import pallas as pl


def kernel(x, mask, w0, w1, b0, b1):
    raise NotImplementedError("write your pallas kernel here")


# speedup vs baseline: 1.0028x; 1.0028x over previous
import jax
import jax.numpy as jnp
from jax.experimental import pallas as pl
from jax.experimental.pallas import tpu as pltpu


def _round_up(n: int, m: int) -> int:
    return ((n + m - 1) // m) * m


def _make_body(bt: int, ts: int):
    def _body(x_ref, m_ref, w0_ref, b0_ref, w1_ref, b1_ref, o_ref, len_ref):
        s = pl.program_id(1)

        @pl.when(s == 0)
        def _():
            o_ref[...] = jnp.zeros_like(o_ref)
            len_ref[...] = jnp.zeros_like(len_ref)

        xb = x_ref[...].astype(jnp.bfloat16).reshape(bt * ts, -1)
        h = jnp.tanh(
            jnp.dot(xb, w0_ref[...], preferred_element_type=jnp.float32)
            + b0_ref[...]
        )
        h = jnp.tanh(
            jnp.dot(h.astype(jnp.bfloat16), w1_ref[...],
                    preferred_element_type=jnp.float32)
            + b1_ref[...]
        )
        h = h.reshape(bt, ts, h.shape[-1])
        m = m_ref[...].astype(jnp.float32)
        o_ref[...] += jnp.sum(h * m[:, :, None], axis=1)
        len_ref[...] += jnp.sum(m, axis=1, keepdims=True)

        @pl.when(s == pl.num_programs(1) - 1)
        def _():
            inv = 1.0 / jnp.maximum(len_ref[...], 1.0)
            o_ref[...] = o_ref[...] * inv

    return _body


def kernel(x, mask, w0, w1, b0, b1):
    B, S, D_in = x.shape
    H1 = w0.shape[1]
    H2 = w1.shape[1]

    Din_p, H1_p, H2_p = (_round_up(d, 128) for d in (D_in, H1, H2))

    w0p = jnp.zeros((Din_p, H1_p), jnp.bfloat16).at[:D_in, :H1].set(
        w0.astype(jnp.bfloat16))
    w1p = jnp.zeros((H1_p, H2_p), jnp.bfloat16).at[:H1, :H2].set(
        w1.astype(jnp.bfloat16))
    b0p = jnp.zeros((1, H1_p), jnp.float32).at[:, :H1].set(
        b0.reshape(1, -1).astype(jnp.float32))
    b1p = jnp.zeros((1, H2_p), jnp.float32).at[:, :H2].set(
        b1.reshape(1, -1).astype(jnp.float32))

    bt = 8 if B % 8 == 0 else B
    nb = B // bt
    ts = min(512, _round_up(S, 8))
    Sp = _round_up(S, ts)

    xp = x
    mp = mask.astype(jnp.float32)
    if Sp != S or Din_p != D_in:
        xp = jnp.zeros((B, Sp, Din_p), x.dtype).at[:, :S, :D_in].set(x)
        mp = jnp.zeros((B, Sp), jnp.float32).at[:, :S].set(mp)

    grid = (nb, Sp // ts)
    out = pl.pallas_call(
        _make_body(bt, ts),
        out_shape=jax.ShapeDtypeStruct((B, H2_p), jnp.float32),
        grid_spec=pltpu.PrefetchScalarGridSpec(
            num_scalar_prefetch=0,
            grid=grid,
            in_specs=[
                pl.BlockSpec((bt, ts, Din_p), lambda i, s: (i, s, 0)),
                pl.BlockSpec((bt, ts), lambda i, s: (i, s)),
                pl.BlockSpec((Din_p, H1_p), lambda i, s: (0, 0)),
                pl.BlockSpec((1, H1_p), lambda i, s: (0, 0)),
                pl.BlockSpec((H1_p, H2_p), lambda i, s: (0, 0)),
                pl.BlockSpec((1, H2_p), lambda i, s: (0, 0)),
            ],
            out_specs=pl.BlockSpec((bt, H2_p), lambda i, s: (i, 0)),
            scratch_shapes=[pltpu.VMEM((bt, 1), jnp.float32)],
        ),
        compiler_params=pltpu.CompilerParams(
            dimension_semantics=("parallel", "arbitrary"),
            vmem_limit_bytes=56 << 20,
        ),
    )(xp, mp, w0p, b0p, w1p, b1p)
    return out[:, :H2].astype(x.dtype)
